# baseline (speedup 1.0000x reference)
import dataclasses
import functools

import jax
import jax.numpy as jnp
from jax import lax
from jax.experimental import pallas as pl
from jax.experimental.pallas import tpu as pltpu
from jax.experimental.pallas import tpu_sc as plsc

DIM = 64
LANES = 16
NW = 32
NB = 34


def _compiler_params():
    cp = pltpu.CompilerParams(use_tc_tiling_on_sc=False)
    if "needs_layout_passes" in pltpu.CompilerParams.__dataclass_fields__:
        cp = dataclasses.replace(cp, needs_layout_passes=False)
    return cp


def _feature_major(wt):
    v = wt.shape[1]
    vt = (v + 127) // 128
    nb = NB if vt % NB == 0 else vt
    vtb = vt // nb
    dt_n, di_n = DIM // 8, 8

    def tx(w_ref, o_ref):
        o_ref[...] = (
            w_ref[...]
            .reshape(dt_n, di_n, vtb, 128)
            .transpose(0, 2, 1, 3))

    return pl.pallas_call(
        tx,
        grid=(nb,),
        in_specs=[pl.BlockSpec((DIM, vtb * 128), lambda i: (0, i))],
        out_specs=pl.BlockSpec((dt_n, vtb, di_n, 128),
                               lambda i: (0, i, 0, 0)),
        out_shape=jax.ShapeDtypeStruct((dt_n, vt, di_n, 128),
                                       jnp.float32),
    )(wt)


@jax.jit
def kernel(x, weights):
    b, s = x.shape
    scale = jnp.sqrt(jnp.asarray(DIM, dtype=jnp.float32))
    xt = x.T.astype(jnp.int32)
    wt = _feature_major(weights.T)
    vtp = wt.shape[1]
    bt_n, bi_n = b // 128, 128
    dt_n, di_n = DIM // 8, 8

    mesh = plsc.VectorSubcoreMesh(core_axis_name="core",
                                  subcore_axis_name="subcore")

    @functools.partial(
        pl.kernel,
        out_type=jax.ShapeDtypeStruct((s, dt_n, bt_n, di_n, bi_n),
                                      jnp.float32),
        mesh=mesh,
        scratch_types=[
            pltpu.VMEM((vtp, 128), jnp.float32),
            pltpu.VMEM((2, b), jnp.int32),
            pltpu.VMEM((2, bt_n, bi_n), jnp.float32),
            pltpu.VMEM_SHARED((s, b), jnp.int32),
            pltpu.SemaphoreType.DMA,
            pltpu.SemaphoreType.DMA,
            pltpu.SemaphoreType.DMA,
        ],
        compiler_params=_compiler_params(),
    )
    def embed(wt_hbm, xt_hbm, o_hbm, row_v, idx_v, out_v, idx_sp, sem_r,
              sem_i, sem_o):
        sid = lax.axis_index("subcore")
        wid = sid * 2 + lax.axis_index("core")

        rows_per = s // 10

        @pl.when(sid < 10)
        def _():
            first = sid * rows_per
            pltpu.async_copy(xt_hbm.at[pl.ds(first, rows_per)],
                             idx_sp.at[pl.ds(first, rows_per)], sem_i).wait()

        plsc.subcore_barrier()

        def compute(sb, ob):
            @plsc.parallel_loop(0, bt_n, 1, unroll=8)
            def _(bt):
                for j in range(bi_n // LANES):
                    iv = idx_v[sb, pl.ds(bt * bi_n + j * LANES, LANES)]
                    hi = lax.shift_right_logical(iv, 7)
                    lo = lax.bitwise_and(iv, 127)
                    out_v[ob, bt, pl.ds(j * LANES, LANES)] = (
                        plsc.load_gather(row_v, [hi, lo]) * scale)

        def round_(d):
            dt = d // di_n
            di = lax.rem(d, di_n)
            pltpu.async_copy(wt_hbm.at[dt, :, di, :], row_v, sem_r).wait()
            pltpu.async_copy(idx_sp.at[0], idx_v.at[0], sem_i).wait()

            @pl.loop(0, s, step=2)
            def _(s0):
                for ph in range(2):
                    si = s0 + ph
                    nxt = pltpu.make_async_copy(
                        idx_sp.at[si + 1], idx_v.at[1 - ph], sem_i)

                    @pl.when(si + 1 < s)
                    def _():
                        nxt.start()

                    @pl.when(si >= 2)
                    def _():
                        pltpu.make_async_copy(
                            out_v.at[ph],
                            o_hbm.at[si - 2, dt, :, di, :], sem_o).wait()

                    compute(ph, ph)

                    pltpu.make_async_copy(
                        out_v.at[ph], o_hbm.at[si, dt, :, di, :],
                        sem_o).start()

                    @pl.when(si + 1 < s)
                    def _():
                        pltpu.make_async_copy(
                            idx_sp.at[si + 1], idx_v.at[1 - ph],
                            sem_i).wait()

            for ph in range(2):
                pltpu.make_async_copy(
                    out_v.at[ph], o_hbm.at[s - 2 + ph, dt, :, di, :],
                    sem_o).wait()

        round_(wid)
        round_(wid + NW)

    o5 = embed(wt, xt)
    return o5.transpose(2, 4, 0, 1, 3).reshape(b, s, DIM)


# speedup vs baseline: 1.0804x; 1.0804x over previous
import dataclasses
import functools

import jax
import jax.numpy as jnp
from jax import lax
from jax.experimental import pallas as pl
from jax.experimental.pallas import tpu as pltpu
from jax.experimental.pallas import tpu_sc as plsc

DIM = 64
LANES = 16
NW = 32
NB = 17


def _compiler_params():
    cp = pltpu.CompilerParams(use_tc_tiling_on_sc=False)
    if "needs_layout_passes" in pltpu.CompilerParams.__dataclass_fields__:
        cp = dataclasses.replace(cp, needs_layout_passes=False)
    return cp


def _feature_major(wt):
    v = wt.shape[1]
    vt = (v + 127) // 128
    nb = NB if vt % NB == 0 else vt
    vtb = vt // nb
    dt_n, di_n = DIM // 8, 8

    def tx(w_ref, o_ref):
        o_ref[...] = (
            w_ref[...]
            .reshape(dt_n, di_n, vtb, 128)
            .transpose(0, 2, 1, 3))

    return pl.pallas_call(
        tx,
        grid=(nb,),
        in_specs=[pl.BlockSpec((DIM, vtb * 128), lambda i: (0, i))],
        out_specs=pl.BlockSpec((dt_n, vtb, di_n, 128),
                               lambda i: (0, i, 0, 0)),
        out_shape=jax.ShapeDtypeStruct((dt_n, vt, di_n, 128),
                                       jnp.float32),
    )(wt)


@jax.jit
def kernel(x, weights):
    b, s = x.shape
    scale = jnp.sqrt(jnp.asarray(DIM, dtype=jnp.float32))
    xt = x.T.astype(jnp.int32)
    wt = _feature_major(weights.T)
    vtp = wt.shape[1]
    bt_n, bi_n = b // 128, 128
    dt_n, di_n = DIM // 8, 8

    mesh = plsc.VectorSubcoreMesh(core_axis_name="core",
                                  subcore_axis_name="subcore")

    @functools.partial(
        pl.kernel,
        out_type=jax.ShapeDtypeStruct((s, dt_n, bt_n, di_n, bi_n),
                                      jnp.float32),
        mesh=mesh,
        scratch_types=[
            pltpu.VMEM((vtp, 128), jnp.float32),
            pltpu.VMEM((2, b), jnp.int32),
            pltpu.VMEM((2, bt_n, bi_n), jnp.float32),
            pltpu.VMEM_SHARED((s, b), jnp.int32),
            pltpu.SemaphoreType.DMA,
            pltpu.SemaphoreType.DMA,
            pltpu.SemaphoreType.DMA,
        ],
        compiler_params=_compiler_params(),
    )
    def embed(wt_hbm, xt_hbm, o_hbm, row_v, idx_v, out_v, idx_sp, sem_r,
              sem_i, sem_o):
        sid = lax.axis_index("subcore")
        wid = sid * 2 + lax.axis_index("core")

        rows_per = s // 10

        @pl.when(sid < 10)
        def _():
            first = sid * rows_per
            pltpu.async_copy(xt_hbm.at[pl.ds(first, rows_per)],
                             idx_sp.at[pl.ds(first, rows_per)], sem_i).wait()

        plsc.subcore_barrier()

        def compute(sb, ob):
            @plsc.parallel_loop(0, bt_n, 1, unroll=8)
            def _(bt):
                for j in range(bi_n // LANES):
                    iv = idx_v[sb, pl.ds(bt * bi_n + j * LANES, LANES)]
                    hi = lax.shift_right_logical(iv, 7)
                    lo = lax.bitwise_and(iv, 127)
                    out_v[ob, bt, pl.ds(j * LANES, LANES)] = (
                        plsc.load_gather(row_v, [hi, lo]) * scale)

        def round_(d):
            dt = d // di_n
            di = lax.rem(d, di_n)
            pltpu.async_copy(wt_hbm.at[dt, :, di, :], row_v, sem_r).wait()
            pltpu.async_copy(idx_sp.at[0], idx_v.at[0], sem_i).wait()

            @pl.loop(0, s, step=2)
            def _(s0):
                for ph in range(2):
                    si = s0 + ph
                    nxt = pltpu.make_async_copy(
                        idx_sp.at[si + 1], idx_v.at[1 - ph], sem_i)

                    @pl.when(si + 1 < s)
                    def _():
                        nxt.start()

                    @pl.when(si >= 2)
                    def _():
                        pltpu.make_async_copy(
                            out_v.at[ph],
                            o_hbm.at[si - 2, dt, :, di, :], sem_o).wait()

                    compute(ph, ph)

                    pltpu.make_async_copy(
                        out_v.at[ph], o_hbm.at[si, dt, :, di, :],
                        sem_o).start()

                    @pl.when(si + 1 < s)
                    def _():
                        pltpu.make_async_copy(
                            idx_sp.at[si + 1], idx_v.at[1 - ph],
                            sem_i).wait()

            for ph in range(2):
                pltpu.make_async_copy(
                    out_v.at[ph], o_hbm.at[s - 2 + ph, dt, :, di, :],
                    sem_o).wait()

        round_(wid)
        round_(wid + NW)

    o5 = embed(wt, xt)
    return o5.transpose(2, 4, 0, 1, 3).reshape(b, s, DIM)
